# baseline (speedup 1.0000x reference)
import functools

import jax
import jax.numpy as jnp
from jax import lax
from jax.experimental import pallas as pl
from jax.experimental.pallas import tpu as pltpu
from jax.experimental.pallas import tpu_sc as plsc

_B = 8
_NN = 10000
_NA = 4000
_NG = 2048
_NC = 10
_N = _B * _NN
_E = _N * 32
_KK = 5000

_FC1_BLK = 400
_FC1_GRID = _NN // _FC1_BLK
_FC1_MM_STEPS = _NA // _FC1_BLK

_NW = 32
_E_PER_W = _E // _NW
_CHUNKS = 25
_CHUNK = _E_PER_W // _CHUNKS
_NODES_PER_TILE = _N // 16


def _fc1_body(td_ref, w_ref, m_ref, b_ref, xgt_ref, o_ref):
    i = pl.program_id(0)

    @pl.when(i < _FC1_MM_STEPS)
    def _():
        wm = w_ref[...] * m_ref[...]
        o_ref[...] = lax.dot_general(
            wm, td_ref[...], (((1,), (1,)), ((), ())),
            preferred_element_type=jnp.float32) + b_ref[...]

    @pl.when(i >= _FC1_MM_STEPS)
    def _():
        o_ref[...] = xgt_ref[...]


def _fc1(td, fc1_W, adj_mask, bcol, xgt):
    return pl.pallas_call(
        _fc1_body,
        grid=(_FC1_GRID,),
        in_specs=[
            pl.BlockSpec((_B, _NG), lambda i: (0, 0)),
            pl.BlockSpec((_FC1_BLK, _NG), lambda i: (jnp.minimum(i, _FC1_MM_STEPS - 1), 0)),
            pl.BlockSpec((_FC1_BLK, _NG), lambda i: (jnp.minimum(i, _FC1_MM_STEPS - 1), 0)),
            pl.BlockSpec((_FC1_BLK, 1), lambda i: (jnp.minimum(i, _FC1_MM_STEPS - 1), 0)),
            pl.BlockSpec((_FC1_BLK, _B), lambda i: (i, 0)),
        ],
        out_specs=pl.BlockSpec((_FC1_BLK, _B), lambda i: (i, 0)),
        out_shape=jax.ShapeDtypeStruct((_NN, _B), jnp.float32),
    )(td, fc1_W, adj_mask, bcol, xgt)


def _sc_agg(xflat, ei2, zeros, ones):
    mesh = plsc.VectorSubcoreMesh(core_axis_name="c", subcore_axis_name="s")

    @functools.partial(
        pl.kernel,
        out_type=[
            jax.ShapeDtypeStruct((2, _N), jnp.float32),
            jax.ShapeDtypeStruct((2, _N), jnp.float32),
        ],
        mesh=mesh,
        scratch_types=[
            pltpu.VMEM((_N,), jnp.float32),
            pltpu.VMEM((_CHUNK,), jnp.int32),
            pltpu.VMEM((_CHUNK,), jnp.int32),
            pltpu.VMEM((_CHUNK,), jnp.float32),
            pltpu.VMEM((_CHUNK,), jnp.float32),
            pltpu.VMEM_SHARED((_N,), jnp.float32),
            pltpu.VMEM_SHARED((_N,), jnp.float32),
            pltpu.SemaphoreType.DMA,
        ],
        compiler_params=pltpu.CompilerParams(
            use_tc_tiling_on_sc=False, needs_layout_passes=False),
    )
    def k(x_hbm, ei_hbm, z_hbm, o_hbm, agg_hbm, deg_hbm,
          x_tab, src_v, dst_v, val_v, ones_v, agg_sh, deg_sh, sem):
        c = lax.axis_index("c")
        s = lax.axis_index("s")
        w = s * 2 + c

        zslice = pl.ds(s * _NODES_PER_TILE, _NODES_PER_TILE)
        pltpu.sync_copy(z_hbm.at[zslice], agg_sh.at[zslice])
        pltpu.sync_copy(z_hbm.at[zslice], deg_sh.at[zslice])
        pltpu.sync_copy(x_hbm, x_tab)
        pltpu.sync_copy(o_hbm, ones_v)
        plsc.subcore_barrier()

        ebase = w * _E_PER_W

        def chunk(g, carry):
            off = ebase + g * _CHUNK
            pltpu.sync_copy(ei_hbm.at[pl.ds(off, _CHUNK)], src_v)
            pltpu.sync_copy(ei_hbm.at[pl.ds(_E + off, _CHUNK)], dst_v)
            for j in range(_CHUNK // 16):
                idx = src_v[pl.ds(j * 16, 16)]
                val_v[pl.ds(j * 16, 16)] = plsc.load_gather(x_tab, [idx])
            d1 = pltpu.async_copy(val_v, agg_sh.at[dst_v], sem, add=True)
            d2 = pltpu.async_copy(ones_v, deg_sh.at[dst_v], sem, add=True)
            d1.wait()
            d2.wait()
            return carry

        lax.fori_loop(0, _CHUNKS, chunk, 0)
        plsc.subcore_barrier()

        pltpu.sync_copy(agg_sh.at[zslice], agg_hbm.at[c, zslice])
        pltpu.sync_copy(deg_sh.at[zslice], deg_hbm.at[c, zslice])

    return k(xflat, ei2, zeros, ones)


def _finish_body(aggs_ref, degs_ref, xg_ref, consts_ref, w2_ref, b2_ref, o_ref):
    agg = aggs_ref[0] + aggs_ref[1]
    deg = degs_ref[0] + degs_ref[1]
    mean = agg / jnp.maximum(deg, 1.0)
    wr = consts_ref[0, 0]
    wn = consts_ref[0, 1]
    pb = consts_ref[0, 2]
    h = xg_ref[...] * wr + mean * wn + pb
    a = jnp.abs(h)
    ai = lax.bitcast_convert_type(a, jnp.int32)

    t = jnp.zeros((_B, 1), jnp.int32)
    for bit in range(30, -1, -1):
        cand = t | (1 << bit)
        cnt = jnp.sum((ai >= cand).astype(jnp.int32), axis=1, keepdims=True)
        t = jnp.where(cnt >= _KK, cand, t)

    gt = ai > t
    eq = ai == t
    cnt_gt = jnp.sum(gt.astype(jnp.int32), axis=1, keepdims=True)
    need = _KK - cnt_gt
    col = lax.broadcasted_iota(jnp.int32, (_B, _NN), 1)
    m = jnp.zeros((_B, 1), jnp.int32)
    for bit in range(13, -1, -1):
        cand = m | (1 << bit)
        cm = jnp.sum((eq & (col < cand)).astype(jnp.int32), axis=1, keepdims=True)
        m = jnp.where(cm <= need, cand, m)

    mask = gt | (eq & (col < m))
    dense = jnp.where(mask, h, 0.0)
    o_ref[...] = lax.dot_general(
        dense, w2_ref[...], (((1,), (1,)), ((), ())),
        preferred_element_type=jnp.float32) + b2_ref[...]


def _finish(aggs, degs, xg, consts, fc2_W, b2d):
    return pl.pallas_call(
        _finish_body,
        in_specs=[
            pl.BlockSpec(memory_space=pltpu.MemorySpace.VMEM),
            pl.BlockSpec(memory_space=pltpu.MemorySpace.VMEM),
            pl.BlockSpec(memory_space=pltpu.MemorySpace.VMEM),
            pl.BlockSpec(memory_space=pltpu.MemorySpace.SMEM),
            pl.BlockSpec(memory_space=pltpu.MemorySpace.VMEM),
            pl.BlockSpec(memory_space=pltpu.MemorySpace.VMEM),
        ],
        out_specs=pl.BlockSpec(memory_space=pltpu.MemorySpace.VMEM),
        out_shape=jax.ShapeDtypeStruct((_B, _NC), jnp.float32),
    )(aggs, degs, xg, consts, fc2_W, b2d)


def kernel(transcriptomic_data, x, edge_index, batch, fc1_W, fc1_b, adj_mask,
           W_root, W_nb, prop_b, fc2_W, fc2_b):
    xgt = x.reshape(_B, _NN).T
    ei2 = edge_index.reshape(2 * _E)

    x_new_t = _fc1(transcriptomic_data, fc1_W, adj_mask,
                   fc1_b.reshape(_NA, 1), xgt)
    x_new = x_new_t.T
    xflat = x_new.reshape(_N)

    zeros = jnp.zeros((_N,), jnp.float32)
    ones = jnp.ones((_CHUNK,), jnp.float32)
    agg2, deg2 = _sc_agg(xflat, ei2, zeros, ones)
    aggs = agg2.reshape(2, _B, _NN)
    degs = deg2.reshape(2, _B, _NN)

    consts = jnp.stack([W_root.reshape(()), W_nb.reshape(()),
                        prop_b.reshape(())]).reshape(1, 3)
    out = _finish(aggs, degs, x_new, consts, fc2_W, fc2_b.reshape(1, _NC))
    return out


# speedup vs baseline: 1.2626x; 1.2626x over previous
import functools

import jax
import jax.numpy as jnp
from jax import lax
from jax.experimental import pallas as pl
from jax.experimental.pallas import tpu as pltpu
from jax.experimental.pallas import tpu_sc as plsc

_B = 8
_NN = 10000
_NA = 4000
_NG = 2048
_NC = 10
_N = _B * _NN
_E = _N * 32
_KK = 5000

_FC1_BLK = 400
_FC1_GRID = _NN // _FC1_BLK
_FC1_MM_STEPS = _NA // _FC1_BLK

_NW = 32
_E_PER_W = _E // _NW
_CHUNKS = 25
_CHUNK = _E_PER_W // _CHUNKS
_NODES_PER_TILE = _N // 16


def _fc1_body(td_ref, w_ref, m_ref, b_ref, xgt_ref, o_ref):
    i = pl.program_id(0)

    @pl.when(i < _FC1_MM_STEPS)
    def _():
        wm = w_ref[...] * m_ref[...]
        o_ref[...] = lax.dot_general(
            wm, td_ref[...], (((1,), (1,)), ((), ())),
            preferred_element_type=jnp.float32) + b_ref[...]

    @pl.when(i >= _FC1_MM_STEPS)
    def _():
        o_ref[...] = xgt_ref[...]


def _fc1(td, fc1_W, adj_mask, bcol, xgt):
    return pl.pallas_call(
        _fc1_body,
        grid=(_FC1_GRID,),
        in_specs=[
            pl.BlockSpec((_B, _NG), lambda i: (0, 0)),
            pl.BlockSpec((_FC1_BLK, _NG), lambda i: (jnp.minimum(i, _FC1_MM_STEPS - 1), 0)),
            pl.BlockSpec((_FC1_BLK, _NG), lambda i: (jnp.minimum(i, _FC1_MM_STEPS - 1), 0)),
            pl.BlockSpec((_FC1_BLK, 1), lambda i: (jnp.minimum(i, _FC1_MM_STEPS - 1), 0)),
            pl.BlockSpec((_FC1_BLK, _B), lambda i: (i, 0)),
        ],
        out_specs=pl.BlockSpec((_FC1_BLK, _B), lambda i: (i, 0)),
        out_shape=jax.ShapeDtypeStruct((_NN, _B), jnp.float32),
    )(td, fc1_W, adj_mask, bcol, xgt)


def _sc_agg(xflat, ei2, zeros, ones):
    mesh = plsc.VectorSubcoreMesh(core_axis_name="c", subcore_axis_name="s")

    @functools.partial(
        pl.kernel,
        out_type=[
            jax.ShapeDtypeStruct((2, _N), jnp.float32),
            jax.ShapeDtypeStruct((2, _N), jnp.float32),
        ],
        mesh=mesh,
        scratch_types=[
            pltpu.VMEM((_N,), jnp.float32),
            pltpu.VMEM((2, _CHUNK), jnp.int32),
            pltpu.VMEM((2, _CHUNK), jnp.int32),
            pltpu.VMEM((2, _CHUNK), jnp.float32),
            pltpu.VMEM((_CHUNK,), jnp.float32),
            pltpu.VMEM_SHARED((_N,), jnp.float32),
            pltpu.VMEM_SHARED((_N,), jnp.float32),
            pltpu.SemaphoreType.DMA,
            pltpu.SemaphoreType.DMA,
        ],
        compiler_params=pltpu.CompilerParams(
            use_tc_tiling_on_sc=False, needs_layout_passes=False),
    )
    def k(x_hbm, ei_hbm, z_hbm, o_hbm, agg_hbm, deg_hbm,
          x_tab, src_v, dst_v, val_v, ones_v, agg_sh, deg_sh,
          sc_sem0, sc_sem1):
        c = lax.axis_index("c")
        s = lax.axis_index("s")
        w = s * 2 + c
        sc_sems = (sc_sem0, sc_sem1)

        ebase = w * _E_PER_W

        def gather(ph):
            for j in range(_CHUNK // 16):
                idx = src_v[ph, pl.ds(j * 16, 16)]
                val_v[ph, pl.ds(j * 16, 16)] = plsc.load_gather(x_tab, [idx])

        def issue_scatters(ph):
            pltpu.async_copy(val_v.at[ph], agg_sh.at[dst_v.at[ph]],
                             sc_sems[ph], add=True)
            pltpu.async_copy(ones_v, deg_sh.at[dst_v.at[ph]],
                             sc_sems[ph], add=True)

        def drain_scatters(ph):
            pltpu.make_async_copy(val_v.at[ph], agg_sh.at[dst_v.at[ph]],
                                  sc_sems[ph]).wait()
            pltpu.make_async_copy(ones_v, deg_sh.at[dst_v.at[ph]],
                                  sc_sems[ph]).wait()

        zslice = pl.ds(s * _NODES_PER_TILE, _NODES_PER_TILE)
        pltpu.sync_copy(z_hbm.at[zslice], agg_sh.at[zslice])
        pltpu.sync_copy(z_hbm.at[zslice], deg_sh.at[zslice])
        pltpu.sync_copy(x_hbm, x_tab)
        pltpu.sync_copy(o_hbm, ones_v)
        plsc.subcore_barrier()

        def sync_loads(g, ph):
            off = ebase + g * _CHUNK
            pltpu.sync_copy(ei_hbm.at[pl.ds(off, _CHUNK)], src_v.at[ph])
            pltpu.sync_copy(ei_hbm.at[pl.ds(_E + off, _CHUNK)], dst_v.at[ph])

        def pair(p, carry):
            g0 = 2 * p

            @pl.when(p > 0)
            def _():
                drain_scatters(0)
            sync_loads(g0, 0)
            gather(0)
            issue_scatters(0)

            @pl.when(p > 0)
            def _():
                drain_scatters(1)
            sync_loads(g0 + 1, 1)
            gather(1)
            issue_scatters(1)
            return carry

        lax.fori_loop(0, (_CHUNKS - 1) // 2, pair, 0)
        drain_scatters(0)
        sync_loads(_CHUNKS - 1, 0)
        gather(0)
        issue_scatters(0)
        drain_scatters(1)
        drain_scatters(0)
        plsc.subcore_barrier()

        pltpu.sync_copy(agg_sh.at[zslice], agg_hbm.at[c, zslice])
        pltpu.sync_copy(deg_sh.at[zslice], deg_hbm.at[c, zslice])

    return k(xflat, ei2, zeros, ones)


def _finish_body(aggs_ref, degs_ref, xg_ref, consts_ref, w2_ref, b2_ref, o_ref):
    agg = aggs_ref[0] + aggs_ref[1]
    deg = degs_ref[0] + degs_ref[1]
    mean = agg / jnp.maximum(deg, 1.0)
    wr = consts_ref[0, 0]
    wn = consts_ref[0, 1]
    pb = consts_ref[0, 2]
    h = xg_ref[...] * wr + mean * wn + pb
    a = jnp.abs(h)
    ai = lax.bitcast_convert_type(a, jnp.int32)

    t = jnp.zeros((_B, 1), jnp.int32)
    for bit in range(30, -1, -1):
        cand = t | (1 << bit)
        cnt = jnp.sum((ai >= cand).astype(jnp.int32), axis=1, keepdims=True)
        t = jnp.where(cnt >= _KK, cand, t)

    gt = ai > t
    eq = ai == t
    cnt_gt = jnp.sum(gt.astype(jnp.int32), axis=1, keepdims=True)
    need = _KK - cnt_gt
    col = lax.broadcasted_iota(jnp.int32, (_B, _NN), 1)
    m = jnp.zeros((_B, 1), jnp.int32)
    for bit in range(13, -1, -1):
        cand = m | (1 << bit)
        cm = jnp.sum((eq & (col < cand)).astype(jnp.int32), axis=1, keepdims=True)
        m = jnp.where(cm <= need, cand, m)

    mask = gt | (eq & (col < m))
    dense = jnp.where(mask, h, 0.0)
    o_ref[...] = lax.dot_general(
        dense, w2_ref[...], (((1,), (1,)), ((), ())),
        preferred_element_type=jnp.float32) + b2_ref[...]


def _finish(aggs, degs, xg, consts, fc2_W, b2d):
    return pl.pallas_call(
        _finish_body,
        in_specs=[
            pl.BlockSpec(memory_space=pltpu.MemorySpace.VMEM),
            pl.BlockSpec(memory_space=pltpu.MemorySpace.VMEM),
            pl.BlockSpec(memory_space=pltpu.MemorySpace.VMEM),
            pl.BlockSpec(memory_space=pltpu.MemorySpace.SMEM),
            pl.BlockSpec(memory_space=pltpu.MemorySpace.VMEM),
            pl.BlockSpec(memory_space=pltpu.MemorySpace.VMEM),
        ],
        out_specs=pl.BlockSpec(memory_space=pltpu.MemorySpace.VMEM),
        out_shape=jax.ShapeDtypeStruct((_B, _NC), jnp.float32),
    )(aggs, degs, xg, consts, fc2_W, b2d)


def kernel(transcriptomic_data, x, edge_index, batch, fc1_W, fc1_b, adj_mask,
           W_root, W_nb, prop_b, fc2_W, fc2_b):
    xgt = x.reshape(_B, _NN).T
    ei2 = edge_index.reshape(2 * _E)

    x_new_t = _fc1(transcriptomic_data, fc1_W, adj_mask,
                   fc1_b.reshape(_NA, 1), xgt)
    x_new = x_new_t.T
    xflat = x_new.reshape(_N)

    zeros = jnp.zeros((_N,), jnp.float32)
    ones = jnp.ones((_CHUNK,), jnp.float32)
    agg2, deg2 = _sc_agg(xflat, ei2, zeros, ones)
    aggs = agg2.reshape(2, _B, _NN)
    degs = deg2.reshape(2, _B, _NN)

    consts = jnp.stack([W_root.reshape(()), W_nb.reshape(()),
                        prop_b.reshape(())]).reshape(1, 3)
    out = _finish(aggs, degs, x_new, consts, fc2_W, fc2_b.reshape(1, _NC))
    return out


# speedup vs baseline: 1.3550x; 1.0732x over previous
import functools

import jax
import jax.numpy as jnp
from jax import lax
from jax.experimental import pallas as pl
from jax.experimental.pallas import tpu as pltpu
from jax.experimental.pallas import tpu_sc as plsc

_B = 8
_NN = 10000
_NA = 4000
_NG = 2048
_NC = 10
_N = _B * _NN
_E = _N * 32
_KK = 5000

_FC1_BLK = 400
_FC1_GRID = _NN // _FC1_BLK
_FC1_MM_STEPS = _NA // _FC1_BLK

_NW = 32
_E_PER_W = _E // _NW
_CHUNKS = 25
_CHUNK = _E_PER_W // _CHUNKS
_NODES_PER_TILE = _N // 16


def _fc1_body(td_ref, w_ref, m_ref, b_ref, xgt_ref, o_ref):
    i = pl.program_id(0)

    @pl.when(i < _FC1_MM_STEPS)
    def _():
        wm = w_ref[...] * m_ref[...]
        o_ref[...] = lax.dot_general(
            wm, td_ref[...], (((1,), (1,)), ((), ())),
            preferred_element_type=jnp.float32) + b_ref[...]

    @pl.when(i >= _FC1_MM_STEPS)
    def _():
        o_ref[...] = xgt_ref[...]


def _fc1(td, fc1_W, adj_mask, bcol, xgt):
    return pl.pallas_call(
        _fc1_body,
        grid=(_FC1_GRID,),
        in_specs=[
            pl.BlockSpec((_B, _NG), lambda i: (0, 0)),
            pl.BlockSpec((_FC1_BLK, _NG), lambda i: (jnp.minimum(i, _FC1_MM_STEPS - 1), 0)),
            pl.BlockSpec((_FC1_BLK, _NG), lambda i: (jnp.minimum(i, _FC1_MM_STEPS - 1), 0)),
            pl.BlockSpec((_FC1_BLK, 1), lambda i: (jnp.minimum(i, _FC1_MM_STEPS - 1), 0)),
            pl.BlockSpec((_FC1_BLK, _B), lambda i: (i, 0)),
        ],
        out_specs=pl.BlockSpec((_FC1_BLK, _B), lambda i: (i, 0)),
        out_shape=jax.ShapeDtypeStruct((_NN, _B), jnp.float32),
    )(td, fc1_W, adj_mask, bcol, xgt)


def _sc_agg(xflat, ei2, zeros, ones):
    mesh = plsc.VectorSubcoreMesh(core_axis_name="c", subcore_axis_name="s")

    @functools.partial(
        pl.kernel,
        out_type=[
            jax.ShapeDtypeStruct((2, _N), jnp.float32),
            jax.ShapeDtypeStruct((2, _N), jnp.float32),
        ],
        mesh=mesh,
        scratch_types=[
            pltpu.VMEM((_N,), jnp.float32),
            pltpu.VMEM((2, _CHUNK), jnp.int32),
            pltpu.VMEM((2, _CHUNK), jnp.int32),
            pltpu.VMEM((2, _CHUNK), jnp.float32),
            pltpu.VMEM((_CHUNK,), jnp.float32),
            pltpu.VMEM_SHARED((_N,), jnp.float32),
            pltpu.VMEM_SHARED((_N,), jnp.float32),
            pltpu.SemaphoreType.DMA,
            pltpu.SemaphoreType.DMA,
        ],
        compiler_params=pltpu.CompilerParams(
            use_tc_tiling_on_sc=False, needs_layout_passes=False),
    )
    def k(x_hbm, ei_hbm, z_hbm, o_hbm, agg_hbm, deg_hbm,
          x_tab, src_v, dst_v, val_v, ones_v, agg_sh, deg_sh,
          sc_sem0, sc_sem1):
        c = lax.axis_index("c")
        s = lax.axis_index("s")
        w = s * 2 + c
        sc_sems = (sc_sem0, sc_sem1)

        ebase = w * _E_PER_W

        def gather(ph):
            grp = 8
            for j0 in range(0, _CHUNK // 16, grp):
                idxs = [src_v[ph, pl.ds((j0 + k) * 16, 16)] for k in range(grp)]
                vals = [plsc.load_gather(x_tab, [ix]) for ix in idxs]
                for k in range(grp):
                    val_v[ph, pl.ds((j0 + k) * 16, 16)] = vals[k]

        def issue_scatters(ph):
            pltpu.async_copy(val_v.at[ph], agg_sh.at[dst_v.at[ph]],
                             sc_sems[ph], add=True)
            pltpu.async_copy(ones_v, deg_sh.at[dst_v.at[ph]],
                             sc_sems[ph], add=True)

        def drain_scatters(ph):
            pltpu.make_async_copy(val_v.at[ph], agg_sh.at[dst_v.at[ph]],
                                  sc_sems[ph]).wait()
            pltpu.make_async_copy(ones_v, deg_sh.at[dst_v.at[ph]],
                                  sc_sems[ph]).wait()

        zslice = pl.ds(s * _NODES_PER_TILE, _NODES_PER_TILE)
        pltpu.sync_copy(z_hbm.at[zslice], agg_sh.at[zslice])
        pltpu.sync_copy(z_hbm.at[zslice], deg_sh.at[zslice])
        pltpu.sync_copy(x_hbm, x_tab)
        pltpu.sync_copy(o_hbm, ones_v)
        plsc.subcore_barrier()

        def sync_loads(g, ph):
            off = ebase + g * _CHUNK
            pltpu.sync_copy(ei_hbm.at[pl.ds(off, _CHUNK)], src_v.at[ph])
            pltpu.sync_copy(ei_hbm.at[pl.ds(_E + off, _CHUNK)], dst_v.at[ph])

        def pair(p, carry):
            g0 = 2 * p

            @pl.when(p > 0)
            def _():
                drain_scatters(0)
            sync_loads(g0, 0)
            gather(0)
            issue_scatters(0)

            @pl.when(p > 0)
            def _():
                drain_scatters(1)
            sync_loads(g0 + 1, 1)
            gather(1)
            issue_scatters(1)
            return carry

        lax.fori_loop(0, (_CHUNKS - 1) // 2, pair, 0)
        drain_scatters(0)
        sync_loads(_CHUNKS - 1, 0)
        gather(0)
        issue_scatters(0)
        drain_scatters(1)
        drain_scatters(0)
        plsc.subcore_barrier()

        pltpu.sync_copy(agg_sh.at[zslice], agg_hbm.at[c, zslice])
        pltpu.sync_copy(deg_sh.at[zslice], deg_hbm.at[c, zslice])

    return k(xflat, ei2, zeros, ones)


def _finish_body(aggs_ref, degs_ref, xg_ref, consts_ref, w2_ref, b2_ref, o_ref):
    agg = aggs_ref[0] + aggs_ref[1]
    deg = degs_ref[0] + degs_ref[1]
    mean = agg / jnp.maximum(deg, 1.0)
    wr = consts_ref[0, 0]
    wn = consts_ref[0, 1]
    pb = consts_ref[0, 2]
    h = xg_ref[...] * wr + mean * wn + pb
    a = jnp.abs(h)
    ai = lax.bitcast_convert_type(a, jnp.int32)

    t = jnp.zeros((_B, 1), jnp.int32)
    for bit in range(30, -1, -1):
        cand = t | (1 << bit)
        cnt = jnp.sum((ai >= cand).astype(jnp.int32), axis=1, keepdims=True)
        t = jnp.where(cnt >= _KK, cand, t)

    gt = ai > t
    eq = ai == t
    cnt_gt = jnp.sum(gt.astype(jnp.int32), axis=1, keepdims=True)
    need = _KK - cnt_gt
    col = lax.broadcasted_iota(jnp.int32, (_B, _NN), 1)
    m = jnp.zeros((_B, 1), jnp.int32)
    for bit in range(13, -1, -1):
        cand = m | (1 << bit)
        cm = jnp.sum((eq & (col < cand)).astype(jnp.int32), axis=1, keepdims=True)
        m = jnp.where(cm <= need, cand, m)

    mask = gt | (eq & (col < m))
    dense = jnp.where(mask, h, 0.0)
    o_ref[...] = lax.dot_general(
        dense, w2_ref[...], (((1,), (1,)), ((), ())),
        preferred_element_type=jnp.float32) + b2_ref[...]


def _finish(aggs, degs, xg, consts, fc2_W, b2d):
    return pl.pallas_call(
        _finish_body,
        in_specs=[
            pl.BlockSpec(memory_space=pltpu.MemorySpace.VMEM),
            pl.BlockSpec(memory_space=pltpu.MemorySpace.VMEM),
            pl.BlockSpec(memory_space=pltpu.MemorySpace.VMEM),
            pl.BlockSpec(memory_space=pltpu.MemorySpace.SMEM),
            pl.BlockSpec(memory_space=pltpu.MemorySpace.VMEM),
            pl.BlockSpec(memory_space=pltpu.MemorySpace.VMEM),
        ],
        out_specs=pl.BlockSpec(memory_space=pltpu.MemorySpace.VMEM),
        out_shape=jax.ShapeDtypeStruct((_B, _NC), jnp.float32),
    )(aggs, degs, xg, consts, fc2_W, b2d)


def kernel(transcriptomic_data, x, edge_index, batch, fc1_W, fc1_b, adj_mask,
           W_root, W_nb, prop_b, fc2_W, fc2_b):
    xgt = x.reshape(_B, _NN).T
    ei2 = edge_index.reshape(2 * _E)

    x_new_t = _fc1(transcriptomic_data, fc1_W, adj_mask,
                   fc1_b.reshape(_NA, 1), xgt)
    x_new = x_new_t.T
    xflat = x_new.reshape(_N)

    zeros = jnp.zeros((_N,), jnp.float32)
    ones = jnp.ones((_CHUNK,), jnp.float32)
    agg2, deg2 = _sc_agg(xflat, ei2, zeros, ones)
    aggs = agg2.reshape(2, _B, _NN)
    degs = deg2.reshape(2, _B, _NN)

    consts = jnp.stack([W_root.reshape(()), W_nb.reshape(()),
                        prop_b.reshape(())]).reshape(1, 3)
    out = _finish(aggs, degs, x_new, consts, fc2_W, fc2_b.reshape(1, _NC))
    return out


# speedup vs baseline: 1.3749x; 1.0147x over previous
import functools

import jax
import jax.numpy as jnp
from jax import lax
from jax.experimental import pallas as pl
from jax.experimental.pallas import tpu as pltpu
from jax.experimental.pallas import tpu_sc as plsc

_B = 8
_NN = 10000
_NA = 4000
_NG = 2048
_NC = 10
_N = _B * _NN
_E = _N * 32
_KK = 5000

_FC1_BLK = 400
_FC1_GRID = _NN // _FC1_BLK
_FC1_MM_STEPS = _NA // _FC1_BLK

_NW = 32
_E_PER_W = _E // _NW
_CHUNKS = 25
_CHUNK = _E_PER_W // _CHUNKS
_NODES_PER_TILE = _N // 16


def _fc1_body(td_ref, w_ref, m_ref, b_ref, xgt_ref, o_ref):
    i = pl.program_id(0)

    @pl.when(i < _FC1_MM_STEPS)
    def _():
        wm = w_ref[...] * m_ref[...]
        o_ref[...] = lax.dot_general(
            wm, td_ref[...], (((1,), (1,)), ((), ())),
            preferred_element_type=jnp.float32) + b_ref[...]

    @pl.when(i >= _FC1_MM_STEPS)
    def _():
        o_ref[...] = xgt_ref[...]


def _fc1(td, fc1_W, adj_mask, bcol, xgt):
    return pl.pallas_call(
        _fc1_body,
        grid=(_FC1_GRID,),
        in_specs=[
            pl.BlockSpec((_B, _NG), lambda i: (0, 0)),
            pl.BlockSpec((_FC1_BLK, _NG), lambda i: (jnp.minimum(i, _FC1_MM_STEPS - 1), 0)),
            pl.BlockSpec((_FC1_BLK, _NG), lambda i: (jnp.minimum(i, _FC1_MM_STEPS - 1), 0)),
            pl.BlockSpec((_FC1_BLK, 1), lambda i: (jnp.minimum(i, _FC1_MM_STEPS - 1), 0)),
            pl.BlockSpec((_FC1_BLK, _B), lambda i: (i, 0)),
        ],
        out_specs=pl.BlockSpec((_FC1_BLK, _B), lambda i: (i, 0)),
        out_shape=jax.ShapeDtypeStruct((_NN, _B), jnp.float32),
    )(td, fc1_W, adj_mask, bcol, xgt)


def _sc_agg(xflat, ei2, zeros, ones):
    mesh = plsc.VectorSubcoreMesh(core_axis_name="c", subcore_axis_name="s")

    @functools.partial(
        pl.kernel,
        out_type=[
            jax.ShapeDtypeStruct((2, _N), jnp.float32),
            jax.ShapeDtypeStruct((2, _N), jnp.float32),
        ],
        mesh=mesh,
        scratch_types=[
            pltpu.VMEM((_N,), jnp.float32),
            pltpu.VMEM((3, _CHUNK), jnp.int32),
            pltpu.VMEM((3, _CHUNK), jnp.int32),
            pltpu.VMEM((3, _CHUNK), jnp.float32),
            pltpu.VMEM((_CHUNK,), jnp.float32),
            pltpu.VMEM_SHARED((_N,), jnp.float32),
            pltpu.VMEM_SHARED((_N,), jnp.float32),
            pltpu.SemaphoreType.DMA,
            pltpu.SemaphoreType.DMA,
            pltpu.SemaphoreType.DMA,
            pltpu.SemaphoreType.DMA,
            pltpu.SemaphoreType.DMA,
            pltpu.SemaphoreType.DMA,
        ],
        compiler_params=pltpu.CompilerParams(
            use_tc_tiling_on_sc=False, needs_layout_passes=False),
    )
    def k(x_hbm, ei_hbm, z_hbm, o_hbm, agg_hbm, deg_hbm,
          x_tab, src_v, dst_v, val_v, ones_v, agg_sh, deg_sh,
          ld0, ld1, ld2, sc0, sc1, sc2):
        c = lax.axis_index("c")
        s = lax.axis_index("s")
        w = s * 2 + c
        ld_sems = (ld0, ld1, ld2)
        sc_sems = (sc0, sc1, sc2)

        ebase = w * _E_PER_W

        def issue_loads(g, q):
            off = ebase + g * _CHUNK
            pltpu.async_copy(ei_hbm.at[pl.ds(off, _CHUNK)], src_v.at[q],
                             ld_sems[q])
            pltpu.async_copy(ei_hbm.at[pl.ds(_E + off, _CHUNK)], dst_v.at[q],
                             ld_sems[q])

        def wait_loads(q):
            pltpu.make_async_copy(ei_hbm.at[pl.ds(0, _CHUNK)], src_v.at[q],
                                  ld_sems[q]).wait()
            pltpu.make_async_copy(ei_hbm.at[pl.ds(0, _CHUNK)], dst_v.at[q],
                                  ld_sems[q]).wait()

        def gather(ph):
            grp = 8
            for j0 in range(0, _CHUNK // 16, grp):
                idxs = [src_v[ph, pl.ds((j0 + k) * 16, 16)] for k in range(grp)]
                vals = [plsc.load_gather(x_tab, [ix]) for ix in idxs]
                for k in range(grp):
                    val_v[ph, pl.ds((j0 + k) * 16, 16)] = vals[k]

        def issue_scatters(ph):
            pltpu.async_copy(val_v.at[ph], agg_sh.at[dst_v.at[ph]],
                             sc_sems[ph], add=True)
            pltpu.async_copy(ones_v, deg_sh.at[dst_v.at[ph]],
                             sc_sems[ph], add=True)

        def drain_scatters(ph):
            pltpu.make_async_copy(val_v.at[ph], agg_sh.at[dst_v.at[ph]],
                                  sc_sems[ph]).wait()
            pltpu.make_async_copy(ones_v, deg_sh.at[dst_v.at[ph]],
                                  sc_sems[ph]).wait()

        issue_loads(0, 0)

        zslice = pl.ds(s * _NODES_PER_TILE, _NODES_PER_TILE)
        pltpu.sync_copy(z_hbm.at[zslice], agg_sh.at[zslice])
        pltpu.sync_copy(z_hbm.at[zslice], deg_sh.at[zslice])
        pltpu.sync_copy(x_hbm, x_tab)
        pltpu.sync_copy(o_hbm, ones_v)
        plsc.subcore_barrier()

        def group(p, carry):
            for q in range(3):
                g = 3 * p + q

                wait_loads(q)

                @pl.when(g >= 2)
                def _():
                    drain_scatters((q + 1) % 3)

                @pl.when(g + 1 < _CHUNKS)
                def _():
                    issue_loads(g + 1, (q + 1) % 3)

                gather(q)
                issue_scatters(q)
            return carry

        lax.fori_loop(0, (_CHUNKS - 1) // 3, group, 0)
        wait_loads(0)
        drain_scatters(1)
        gather(0)
        issue_scatters(0)
        drain_scatters(2)
        drain_scatters(0)
        plsc.subcore_barrier()

        pltpu.sync_copy(agg_sh.at[zslice], agg_hbm.at[c, zslice])
        pltpu.sync_copy(deg_sh.at[zslice], deg_hbm.at[c, zslice])

    return k(xflat, ei2, zeros, ones)


def _finish_body(aggs_ref, degs_ref, xg_ref, consts_ref, w2_ref, b2_ref, o_ref):
    agg = aggs_ref[0] + aggs_ref[1]
    deg = degs_ref[0] + degs_ref[1]
    mean = agg / jnp.maximum(deg, 1.0)
    wr = consts_ref[0, 0]
    wn = consts_ref[0, 1]
    pb = consts_ref[0, 2]
    h = xg_ref[...] * wr + mean * wn + pb
    a = jnp.abs(h)
    ai = lax.bitcast_convert_type(a, jnp.int32)

    t = jnp.zeros((_B, 1), jnp.int32)
    for bit in range(30, -1, -1):
        cand = t | (1 << bit)
        cnt = jnp.sum((ai >= cand).astype(jnp.int32), axis=1, keepdims=True)
        t = jnp.where(cnt >= _KK, cand, t)

    gt = ai > t
    eq = ai == t
    cnt_gt = jnp.sum(gt.astype(jnp.int32), axis=1, keepdims=True)
    need = _KK - cnt_gt
    col = lax.broadcasted_iota(jnp.int32, (_B, _NN), 1)
    m = jnp.zeros((_B, 1), jnp.int32)
    for bit in range(13, -1, -1):
        cand = m | (1 << bit)
        cm = jnp.sum((eq & (col < cand)).astype(jnp.int32), axis=1, keepdims=True)
        m = jnp.where(cm <= need, cand, m)

    mask = gt | (eq & (col < m))
    dense = jnp.where(mask, h, 0.0)
    o_ref[...] = lax.dot_general(
        dense, w2_ref[...], (((1,), (1,)), ((), ())),
        preferred_element_type=jnp.float32) + b2_ref[...]


def _finish(aggs, degs, xg, consts, fc2_W, b2d):
    return pl.pallas_call(
        _finish_body,
        in_specs=[
            pl.BlockSpec(memory_space=pltpu.MemorySpace.VMEM),
            pl.BlockSpec(memory_space=pltpu.MemorySpace.VMEM),
            pl.BlockSpec(memory_space=pltpu.MemorySpace.VMEM),
            pl.BlockSpec(memory_space=pltpu.MemorySpace.SMEM),
            pl.BlockSpec(memory_space=pltpu.MemorySpace.VMEM),
            pl.BlockSpec(memory_space=pltpu.MemorySpace.VMEM),
        ],
        out_specs=pl.BlockSpec(memory_space=pltpu.MemorySpace.VMEM),
        out_shape=jax.ShapeDtypeStruct((_B, _NC), jnp.float32),
    )(aggs, degs, xg, consts, fc2_W, b2d)


def kernel(transcriptomic_data, x, edge_index, batch, fc1_W, fc1_b, adj_mask,
           W_root, W_nb, prop_b, fc2_W, fc2_b):
    xgt = x.reshape(_B, _NN).T
    ei2 = edge_index.reshape(2 * _E)

    x_new_t = _fc1(transcriptomic_data, fc1_W, adj_mask,
                   fc1_b.reshape(_NA, 1), xgt)
    x_new = x_new_t.T
    xflat = x_new.reshape(_N)

    zeros = jnp.zeros((_N,), jnp.float32)
    ones = jnp.ones((_CHUNK,), jnp.float32)
    agg2, deg2 = _sc_agg(xflat, ei2, zeros, ones)
    aggs = agg2.reshape(2, _B, _NN)
    degs = deg2.reshape(2, _B, _NN)

    consts = jnp.stack([W_root.reshape(()), W_nb.reshape(()),
                        prop_b.reshape(())]).reshape(1, 3)
    out = _finish(aggs, degs, x_new, consts, fc2_W, fc2_b.reshape(1, _NC))
    return out


# speedup vs baseline: 1.5112x; 1.0991x over previous
import functools

import jax
import jax.numpy as jnp
from jax import lax
from jax.experimental import pallas as pl
from jax.experimental.pallas import tpu as pltpu
from jax.experimental.pallas import tpu_sc as plsc

_B = 8
_NN = 10000
_NA = 4000
_NG = 2048
_NC = 10
_N = _B * _NN
_E = _N * 32
_KK = 5000

_FC1_BLK = 400
_FC1_GRID = _NN // _FC1_BLK
_FC1_MM_STEPS = _NA // _FC1_BLK

_NW = 32
_E_PER_W = _E // _NW
_CHUNKS = 25
_CHUNK = _E_PER_W // _CHUNKS
_NODES_PER_TILE = _N // 16


def _fc1_body(td_ref, w_ref, m_ref, b_ref, o_ref):
    wm = w_ref[...] * m_ref[...]
    o_ref[...] = lax.dot_general(
        wm, td_ref[...], (((1,), (1,)), ((), ())),
        preferred_element_type=jnp.float32) + b_ref[...]


def _fc1(td, fc1_W, adj_mask, bcol):
    return pl.pallas_call(
        _fc1_body,
        grid=(_FC1_MM_STEPS,),
        in_specs=[
            pl.BlockSpec((_B, _NG), lambda i: (0, 0)),
            pl.BlockSpec((_FC1_BLK, _NG), lambda i: (i, 0)),
            pl.BlockSpec((_FC1_BLK, _NG), lambda i: (i, 0)),
            pl.BlockSpec((_FC1_BLK, 1), lambda i: (i, 0)),
        ],
        out_specs=pl.BlockSpec((_FC1_BLK, _B), lambda i: (i, 0)),
        out_shape=jax.ShapeDtypeStruct((_NA, _B), jnp.float32),
    )(td, fc1_W, adj_mask, bcol)


def _sc_agg(xflat, ei2, zeros, ones):
    mesh = plsc.VectorSubcoreMesh(core_axis_name="c", subcore_axis_name="s")

    @functools.partial(
        pl.kernel,
        out_type=[
            jax.ShapeDtypeStruct((2, _B, _NN), jnp.float32),
            jax.ShapeDtypeStruct((2, _B, _NN), jnp.float32),
        ],
        mesh=mesh,
        scratch_types=[
            pltpu.VMEM((_N,), jnp.float32),
            pltpu.VMEM((3, _CHUNK), jnp.int32),
            pltpu.VMEM((3, _CHUNK), jnp.int32),
            pltpu.VMEM((3, _CHUNK), jnp.float32),
            pltpu.VMEM((_CHUNK,), jnp.float32),
            pltpu.VMEM_SHARED((_N,), jnp.float32),
            pltpu.VMEM_SHARED((_N,), jnp.float32),
            pltpu.SemaphoreType.DMA,
            pltpu.SemaphoreType.DMA,
            pltpu.SemaphoreType.DMA,
            pltpu.SemaphoreType.DMA,
            pltpu.SemaphoreType.DMA,
            pltpu.SemaphoreType.DMA,
        ],
        compiler_params=pltpu.CompilerParams(
            use_tc_tiling_on_sc=False, needs_layout_passes=False),
    )
    def k(x_hbm, ei_hbm, z_hbm, o_hbm, agg_hbm, deg_hbm,
          x_tab, src_v, dst_v, val_v, ones_v, agg_sh, deg_sh,
          ld0, ld1, ld2, sc0, sc1, sc2):
        c = lax.axis_index("c")
        s = lax.axis_index("s")
        w = s * 2 + c
        ld_sems = (ld0, ld1, ld2)
        sc_sems = (sc0, sc1, sc2)

        ebase = w * _E_PER_W

        def issue_loads(g, q):
            off = ebase + g * _CHUNK
            pltpu.async_copy(ei_hbm.at[pl.ds(off, _CHUNK)], src_v.at[q],
                             ld_sems[q])
            pltpu.async_copy(ei_hbm.at[pl.ds(_E + off, _CHUNK)], dst_v.at[q],
                             ld_sems[q])

        def wait_loads(q):
            pltpu.make_async_copy(ei_hbm.at[pl.ds(0, _CHUNK)], src_v.at[q],
                                  ld_sems[q]).wait()
            pltpu.make_async_copy(ei_hbm.at[pl.ds(0, _CHUNK)], dst_v.at[q],
                                  ld_sems[q]).wait()

        def gather(ph):
            grp = 8
            for j0 in range(0, _CHUNK // 16, grp):
                idxs = [src_v[ph, pl.ds((j0 + k) * 16, 16)] for k in range(grp)]
                vals = [plsc.load_gather(x_tab, [ix]) for ix in idxs]
                for k in range(grp):
                    val_v[ph, pl.ds((j0 + k) * 16, 16)] = vals[k]

        def issue_scatters(ph):
            pltpu.async_copy(val_v.at[ph], agg_sh.at[dst_v.at[ph]],
                             sc_sems[ph], add=True)
            pltpu.async_copy(ones_v, deg_sh.at[dst_v.at[ph]],
                             sc_sems[ph], add=True)

        def drain_scatters(ph):
            pltpu.make_async_copy(val_v.at[ph], agg_sh.at[dst_v.at[ph]],
                                  sc_sems[ph]).wait()
            pltpu.make_async_copy(ones_v, deg_sh.at[dst_v.at[ph]],
                                  sc_sems[ph]).wait()

        issue_loads(0, 0)

        zslice = pl.ds(s * _NODES_PER_TILE, _NODES_PER_TILE)
        pltpu.sync_copy(z_hbm.at[zslice], agg_sh.at[zslice])
        pltpu.sync_copy(z_hbm.at[zslice], deg_sh.at[zslice])
        pltpu.sync_copy(x_hbm, x_tab)
        pltpu.sync_copy(o_hbm, ones_v)
        plsc.subcore_barrier()

        def group(p, carry):
            for q in range(3):
                g = 3 * p + q

                wait_loads(q)

                @pl.when(g >= 2)
                def _():
                    drain_scatters((q + 1) % 3)

                @pl.when(g + 1 < _CHUNKS)
                def _():
                    issue_loads(g + 1, (q + 1) % 3)

                gather(q)
                issue_scatters(q)
            return carry

        lax.fori_loop(0, (_CHUNKS - 1) // 3, group, 0)
        wait_loads(0)
        drain_scatters(1)
        gather(0)
        issue_scatters(0)
        drain_scatters(2)
        drain_scatters(0)
        plsc.subcore_barrier()

        g0 = s // 2
        half = pl.ds((s % 2) * _NODES_PER_TILE, _NODES_PER_TILE)
        pltpu.sync_copy(agg_sh.at[zslice], agg_hbm.at[c, g0, half])
        pltpu.sync_copy(deg_sh.at[zslice], deg_hbm.at[c, g0, half])

    return k(xflat, ei2, zeros, ones)


def _finish_body(aggs_ref, degs_ref, xg_ref, consts_ref, w2_ref, b2_ref, o_ref):
    agg = aggs_ref[0] + aggs_ref[1]
    deg = degs_ref[0] + degs_ref[1]
    mean = agg / jnp.maximum(deg, 1.0)
    wr = consts_ref[0, 0]
    wn = consts_ref[0, 1]
    pb = consts_ref[0, 2]
    h = xg_ref[...] * wr + mean * wn + pb
    a = jnp.abs(h)
    ai = lax.bitcast_convert_type(a, jnp.int32)

    t = jnp.zeros((_B, 1), jnp.int32)
    for bit in range(30, -1, -1):
        cand = t | (1 << bit)
        cnt = jnp.sum((ai >= cand).astype(jnp.int32), axis=1, keepdims=True)
        t = jnp.where(cnt >= _KK, cand, t)

    gt = ai > t
    eq = ai == t
    cnt_gt = jnp.sum(gt.astype(jnp.int32), axis=1, keepdims=True)
    need = _KK - cnt_gt
    col = lax.broadcasted_iota(jnp.int32, (_B, _NN), 1)
    m = jnp.zeros((_B, 1), jnp.int32)
    for bit in range(13, -1, -1):
        cand = m | (1 << bit)
        cm = jnp.sum((eq & (col < cand)).astype(jnp.int32), axis=1, keepdims=True)
        m = jnp.where(cm <= need, cand, m)

    mask = gt | (eq & (col < m))
    dense = jnp.where(mask, h, 0.0)
    o_ref[...] = lax.dot_general(
        dense, w2_ref[...], (((1,), (1,)), ((), ())),
        preferred_element_type=jnp.float32) + b2_ref[...]


def _finish(aggs, degs, xg, consts, fc2_W, b2d):
    return pl.pallas_call(
        _finish_body,
        in_specs=[
            pl.BlockSpec(memory_space=pltpu.MemorySpace.VMEM),
            pl.BlockSpec(memory_space=pltpu.MemorySpace.VMEM),
            pl.BlockSpec(memory_space=pltpu.MemorySpace.VMEM),
            pl.BlockSpec(memory_space=pltpu.MemorySpace.SMEM),
            pl.BlockSpec(memory_space=pltpu.MemorySpace.VMEM),
            pl.BlockSpec(memory_space=pltpu.MemorySpace.VMEM),
        ],
        out_specs=pl.BlockSpec(memory_space=pltpu.MemorySpace.VMEM),
        out_shape=jax.ShapeDtypeStruct((_B, _NC), jnp.float32),
    )(aggs, degs, xg, consts, fc2_W, b2d)


def kernel(transcriptomic_data, x, edge_index, batch, fc1_W, fc1_b, adj_mask,
           W_root, W_nb, prop_b, fc2_W, fc2_b):
    xg = x.reshape(_B, _NN)
    ei2 = edge_index.reshape(2 * _E)

    emb_t = _fc1(transcriptomic_data, fc1_W, adj_mask,
                 fc1_b.reshape(_NA, 1))
    x_new = jnp.concatenate([emb_t.T, xg[:, _NA:]], axis=1)
    xflat = x_new.reshape(_N)

    zeros = jnp.zeros((_N,), jnp.float32)
    ones = jnp.ones((_CHUNK,), jnp.float32)
    aggs, degs = _sc_agg(xflat, ei2, zeros, ones)

    consts = jnp.stack([W_root.reshape(()), W_nb.reshape(()),
                        prop_b.reshape(())]).reshape(1, 3)
    out = _finish(aggs, degs, x_new, consts, fc2_W, fc2_b.reshape(1, _NC))
    return out
